# baseline (speedup 1.0000x reference)
import functools

import jax
import jax.numpy as jnp
from jax import lax
from jax.experimental import pallas as pl
from jax.experimental.pallas import tpu as pltpu
from jax.experimental.pallas import tpu_sc as plsc


@functools.lru_cache(maxsize=None)
def _make_pooling_kernel(B, L, V, D):
    info = plsc.get_sparse_core_info()
    NC, NS, NL = info.num_cores, info.num_subcores, info.num_lanes
    NW = NC * NS
    assert B % NW == 0 and D % NL == 0
    b_per_w = B // NW
    ND = D // NL
    inv_l = 1.0 / L

    chunks = []
    off = 0
    while off < L:
        sz = min(128, L - off)
        chunks.append((off, sz))
        off += sz

    mesh = plsc.VectorSubcoreMesh(core_axis_name="c", subcore_axis_name="s")

    @functools.partial(
        pl.kernel,
        mesh=mesh,
        compiler_params=pltpu.CompilerParams(use_tc_tiling_on_sc=False),
        out_type=jax.ShapeDtypeStruct((B, D), jnp.float32),
        scratch_types=[
            pltpu.VMEM((L,), jnp.int32),
            pltpu.VMEM((L, D), jnp.float32),
            pltpu.VMEM((b_per_w, D), jnp.float32),
            pltpu.SemaphoreType.DMA,
        ],
    )
    def pooled(x_hbm, emb_hbm, out_hbm, idx_v, rows_v, out_v, sem):
        wid = lax.axis_index("s") * NC + lax.axis_index("c")
        base = wid * b_per_w

        def b_body(b, carry):
            pltpu.sync_copy(x_hbm.at[base + b], idx_v)
            copies = [
                pltpu.async_copy(
                    emb_hbm.at[idx_v.at[pl.ds(off, sz)]],
                    rows_v.at[pl.ds(off, sz)],
                    sem,
                )
                for off, sz in chunks
            ]
            for cp in copies:
                cp.wait()

            def j_body(j, accs):
                return tuple(
                    accs[d] + rows_v[j, pl.ds(d * NL, NL)] for d in range(ND)
                )

            accs = lax.fori_loop(
                0, L, j_body,
                tuple(jnp.zeros((NL,), jnp.float32) for _ in range(ND)),
            )
            for d in range(ND):
                out_v[b, pl.ds(d * NL, NL)] = accs[d] * inv_l
            return carry

        lax.fori_loop(0, b_per_w, b_body, 0)
        pltpu.sync_copy(out_v, out_hbm.at[pl.ds(base, b_per_w)])

    return pooled


def kernel(x, embedding):
    B, L = x.shape
    V, D = embedding.shape
    pooled = _make_pooling_kernel(B, L, V, D)
    return pooled(x.astype(jnp.int32), embedding)


# speedup vs baseline: 1.2534x; 1.2534x over previous
import functools

import jax
import jax.numpy as jnp
from jax import lax
from jax.experimental import pallas as pl
from jax.experimental.pallas import tpu as pltpu
from jax.experimental.pallas import tpu_sc as plsc


@functools.lru_cache(maxsize=None)
def _make_pooling_kernel(B, L, V, D):
    info = plsc.get_sparse_core_info()
    NC, NS, NL = info.num_cores, info.num_subcores, info.num_lanes
    NW = NC * NS
    assert B % NW == 0 and D % NL == 0
    b_per_w = B // NW
    ND = D // NL
    inv_l = 1.0 / L

    chunks = []
    off = 0
    while off < L:
        sz = min(128, L - off)
        chunks.append((off, sz))
        off += sz

    U = 8
    while L % U:
        U -= 1

    mesh = plsc.VectorSubcoreMesh(core_axis_name="c", subcore_axis_name="s")

    @functools.partial(
        pl.kernel,
        mesh=mesh,
        compiler_params=pltpu.CompilerParams(use_tc_tiling_on_sc=False),
        out_type=jax.ShapeDtypeStruct((B, D), jnp.float32),
        scratch_types=[
            pltpu.VMEM((b_per_w, L), jnp.int32),
            pltpu.VMEM((2, L, D), jnp.float32),
            pltpu.VMEM((b_per_w, D), jnp.float32),
            pltpu.SemaphoreType.DMA,
            pltpu.SemaphoreType.DMA,
        ],
    )
    def pooled(x_hbm, emb_hbm, out_hbm, idx_v, rows_v, out_v, sem0, sem1):
        wid = lax.axis_index("s") * NC + lax.axis_index("c")
        base = wid * b_per_w
        sems = (sem0, sem1)

        pltpu.sync_copy(x_hbm.at[pl.ds(base, b_per_w)], idx_v)

        def gather_descs(b, slot):
            return [
                pltpu.make_async_copy(
                    emb_hbm.at[idx_v.at[b, pl.ds(off, sz)]],
                    rows_v.at[slot].at[pl.ds(off, sz)],
                    sems[slot],
                )
                for off, sz in chunks
            ]

        def issue(b, slot):
            for cp in gather_descs(b, slot):
                cp.start()

        def drain(b, slot):
            for cp in gather_descs(b, slot):
                cp.wait()

        def accum_row(slot, b):
            def j_body(j, accs):
                new = list(accs)
                for u in range(U):
                    jj = j * U + u
                    for d in range(ND):
                        new[d] = new[d] + rows_v[slot, jj, pl.ds(d * NL, NL)]
                return tuple(new)

            accs = lax.fori_loop(
                0, L // U, j_body,
                tuple(jnp.zeros((NL,), jnp.float32) for _ in range(ND)),
            )
            for d in range(ND):
                out_v[b, pl.ds(d * NL, NL)] = accs[d] * inv_l

        issue(0, 0)

        def b_body(i, carry):
            b0 = 2 * i
            issue(b0 + 1, 1)
            drain(b0, 0)
            accum_row(0, b0)

            @pl.when(b0 + 2 < b_per_w)
            def _():
                issue(b0 + 2, 0)

            drain(b0 + 1, 1)
            accum_row(1, b0 + 1)
            return carry

        lax.fori_loop(0, b_per_w // 2, b_body, 0)
        pltpu.sync_copy(out_v, out_hbm.at[pl.ds(base, b_per_w)])

    return pooled


def kernel(x, embedding):
    B, L = x.shape
    V, D = embedding.shape
    pooled = _make_pooling_kernel(B, L, V, D)
    return pooled(x.astype(jnp.int32), embedding)
